# Initial kernel scaffold; baseline (speedup 1.0000x reference)
import jax
import jax.numpy as jnp
from jax.experimental import pallas as pl


def kernel(x_user, x_item, edge_index_rates, edge_index_rev, snap, W_rel1_rates, b_rel1_rates, W_root1_rates, W_rel1_rev, b_rel1_rev, W_root1_rev, W_rel2_rates, b_rel2_rates, W_root2_rates, W_rel2_rev, b_rel2_rev, W_root2_rev, W_post, b_post, rel_emb):
    raise NotImplementedError("write your pallas kernel here")


# speedup vs baseline: 28.4085x; 28.4085x over previous
import functools

import jax
import jax.numpy as jnp
from jax import lax
from jax.experimental import pallas as pl
from jax.experimental.pallas import tpu as pltpu
from jax.experimental.pallas import tpu_sc as plsc

N = 5000
NPAD = 5008
E = 160000
NSUB = 16
WIN = 128
NWIN = 80
ET = WIN * NWIN
NBUF = 8
EPAD = ET * NSUB
W = 8
F32 = jnp.float32


def _dense_body(xu_ref, xi_ref, wr1r_ref, wo1r_ref, wr1v_ref, wo1v_ref,
                wr2r_ref, wo2r_ref, wr2v_ref, wo2v_ref, wpost_ref,
                br1r_ref, br1v_ref, br2r_ref, br2v_ref, bpost_ref,
                eir_ref, eiv_ref, pads_ref, padd_ref, rel_ref,
                tab_ref, init_ref, src1_ref, src2_ref, dst_ref,
                ssrc_ref, sdst_ref, relm_ref):
    er = eir_ref[...]
    ev = eiv_ref[...]
    ps = pads_ref[...]
    pd = padd_ref[...]
    sr = jnp.concatenate([er[0], ps])
    dr = jnp.concatenate([er[1], pd])
    sv = jnp.concatenate([ev[0], ps])
    dv = jnp.concatenate([ev[1], pd])
    src1_ref[...] = jnp.stack([sr, sv + NPAD]).reshape(2, NSUB, NWIN, WIN)
    src2_ref[...] = jnp.stack([sr + NPAD, sv]).reshape(2, NSUB, NWIN, WIN)
    dst_ref[...] = jnp.stack([dr, dv]).reshape(2, NSUB, NWIN, WIN)
    ssrc_ref[...] = jnp.stack(
        [(sr + NPAD) * W, sv * W + 2]).reshape(2, NSUB, ET)
    sdst_ref[...] = jnp.stack(
        [dr * W + 2, (dv + NPAD) * W]).reshape(2, NSUB, ET)
    relm_ref[...] = jnp.broadcast_to(
        rel_ref[...][:, :, None], (2, 2, 16)).reshape(2, 32)

    dot = functools.partial(jnp.dot, precision=lax.Precision.HIGHEST,
                            preferred_element_type=F32)
    wp = wpost_ref[...]
    g = dot(wr2v_ref[...], wp)
    t = dot(wo2r_ref[...], wp)
    gp = dot(wr2r_ref[...], wp)
    tp = dot(wo2v_ref[...], wp)
    zpad = jnp.zeros((128, 4), F32)
    gt = jnp.concatenate([g, t, zpad], axis=1)
    tg = jnp.concatenate([tp, gp, zpad], axis=1)
    c_item = dot(br2r_ref[...], wp) + bpost_ref[...]
    c_user = dot(br2v_ref[...], wp) + bpost_ref[...]
    zero2 = jnp.zeros((1, 2), F32)
    zero4 = jnp.zeros((1, 4), F32)
    bias_i = dot(br1r_ref[...], gt) + jnp.concatenate(
        [zero2, c_item, zero4], axis=1)
    bias_u = dot(br1v_ref[...], tg) + jnp.concatenate(
        [c_user, zero2, zero4], axis=1)
    mu = jnp.concatenate([dot(wr1r_ref[...], gt),
                          dot(wo1v_ref[...], tg)], axis=1)
    mi = jnp.concatenate([dot(wr1v_ref[...], tg),
                          dot(wo1r_ref[...], gt)], axis=1)
    ou = dot(xu_ref[...], mu)
    oi = dot(xi_ref[...], mi)
    zrows = jnp.zeros((NPAD - N, W), F32)
    tab_ref[0:N] = ou[:, 0:W]
    tab_ref[N:NPAD] = zrows
    tab_ref[NPAD:NPAD + N] = oi[:, 0:W]
    tab_ref[NPAD + N:] = zrows
    init_ref[0:N] = oi[:, W:] + bias_i
    init_ref[N:NPAD] = zrows
    init_ref[NPAD:NPAD + N] = ou[:, W:] + bias_u
    init_ref[NPAD + N:] = zrows


def _dense(xu, xi, wr1r, wo1r, wr1v, wo1v, wr2r, wo2r, wr2v, wo2v, wpost,
           br1r, br1v, br2r, br2v, bpost, eir, eiv, pads, padd, rel):
    i32 = jnp.int32
    outs = [jax.ShapeDtypeStruct((2 * NPAD, W), F32),
            jax.ShapeDtypeStruct((2 * NPAD, W), F32),
            jax.ShapeDtypeStruct((2, NSUB, NWIN, WIN), i32),
            jax.ShapeDtypeStruct((2, NSUB, NWIN, WIN), i32),
            jax.ShapeDtypeStruct((2, NSUB, NWIN, WIN), i32),
            jax.ShapeDtypeStruct((2, NSUB, ET), i32),
            jax.ShapeDtypeStruct((2, NSUB, ET), i32),
            jax.ShapeDtypeStruct((2, 32), F32)]
    return pl.pallas_call(_dense_body, out_shape=outs)(
        xu, xi, wr1r, wo1r, wr1v, wo1v, wr2r, wo2r, wr2v, wo2v, wpost,
        br1r, br1v, br2r, br2v, bpost, eir, eiv, pads, padd, rel)


_MESH = plsc.VectorSubcoreMesh(core_axis_name="c", subcore_axis_name="s")
_SC_PARAMS = pltpu.CompilerParams(needs_layout_passes=False,
                                  use_tc_tiling_on_sc=False)


@functools.partial(
    pl.kernel, mesh=_MESH,
    out_type=jax.ShapeDtypeStruct((2 * NPAD, W), F32),
    scratch_types=[
        pltpu.VMEM_SHARED((NPAD, W), F32),
        pltpu.VMEM((NWIN, WIN), jnp.int32),
        pltpu.VMEM((NWIN, WIN), jnp.int32),
        pltpu.VMEM((NBUF, WIN, W), F32),
        pltpu.VMEM((NPAD, W), F32),
        pltpu.SemaphoreType.DMA((NBUF,)),
        pltpu.SemaphoreType.DMA((NBUF,)),
    ],
    compiler_params=_SC_PARAMS,
)
def _seg_pass(tab_hbm, init_hbm, src_hbm, dst_hbm, out_hbm,
              acc_sp, sidx_v, didx_v, rows_v, bounce_v, gsem, ssem):
    c = lax.axis_index("c")
    s = lax.axis_index("s")

    @pl.when(s == 0)
    def _stage_init():
        pltpu.sync_copy(init_hbm.at[pl.ds(c * NPAD, NPAD)], bounce_v)
        pltpu.sync_copy(bounce_v, acc_sp)

    pltpu.sync_copy(src_hbm.at[c, s], sidx_v)
    pltpu.sync_copy(dst_hbm.at[c, s], didx_v)
    plsc.subcore_barrier()

    def block(i, carry):
        base = i * NBUF
        gs = [pltpu.async_copy(tab_hbm.at[sidx_v.at[base + k]],
                               rows_v.at[k], gsem.at[k])
              for k in range(NBUF)]
        scs = []
        for k in range(NBUF):
            gs[k].wait()
            scs.append(pltpu.async_copy(
                rows_v.at[k], acc_sp.at[didx_v.at[base + k]], ssem.at[k],
                add=True))
        for cp in scs:
            cp.wait()
        return carry

    lax.fori_loop(0, NWIN // NBUF, block, 0)
    plsc.subcore_barrier()

    @pl.when(s == 0)
    def _flush():
        pltpu.sync_copy(acc_sp, bounce_v)
        pltpu.sync_copy(bounce_v, out_hbm.at[pl.ds(c * NPAD, NPAD)])


ETT = E - (NSUB - 1) * ET


@functools.partial(
    pl.kernel, mesh=_MESH,
    out_type=jax.ShapeDtypeStruct((2 * E,), F32),
    scratch_types=[
        pltpu.VMEM((2 * NPAD, W), F32),
        pltpu.VMEM((ET,), jnp.int32),
        pltpu.VMEM((ET,), jnp.int32),
        pltpu.VMEM((ET,), F32),
        pltpu.VMEM((32,), F32),
    ],
    compiler_params=_SC_PARAMS,
)
def _score(ptab_hbm, src_hbm, dst_hbm, rel_hbm, out_hbm,
           tab_v, sidx_v, didx_v, sbuf_v, rel_v):
    c = lax.axis_index("c")
    s = lax.axis_index("s")
    pltpu.sync_copy(ptab_hbm, tab_v)
    pltpu.sync_copy(src_hbm.at[c, s], sidx_v)
    pltpu.sync_copy(dst_hbm.at[c, s], didx_v)
    pltpu.sync_copy(rel_hbm.at[c], rel_v)
    r0 = rel_v[pl.ds(0, 16)]
    r1 = rel_v[pl.ds(16, 16)]

    def group(g, carry):
        sl = pl.ds(g * 16, 16)
        si = sidx_v[sl]
        di = didx_v[sl]
        sro = lax.shift_right_logical(si, 3)
        sco = jnp.bitwise_and(si, 7)
        dro = lax.shift_right_logical(di, 3)
        dco = jnp.bitwise_and(di, 7)
        a0 = plsc.load_gather(tab_v, [sro, sco])
        a1 = plsc.load_gather(tab_v, [sro, sco + 1])
        b0 = plsc.load_gather(tab_v, [dro, dco])
        b1 = plsc.load_gather(tab_v, [dro, dco + 1])
        sbuf_v[sl] = a0 * b0 * r0 + a1 * b1 * r1
        return carry

    lax.fori_loop(0, ET // 16, group, 0)
    base = c * E + s * ET

    @pl.when(s < NSUB - 1)
    def _full():
        pltpu.sync_copy(sbuf_v, out_hbm.at[pl.ds(base, ET)])

    @pl.when(s == NSUB - 1)
    def _tail():
        pltpu.sync_copy(sbuf_v.at[pl.ds(0, ETT)], out_hbm.at[pl.ds(base, ETT)])


def kernel(x_user, x_item, edge_index_rates, edge_index_rev, snap,
           W_rel1_rates, b_rel1_rates, W_root1_rates,
           W_rel1_rev, b_rel1_rev, W_root1_rev,
           W_rel2_rates, b_rel2_rates, W_root2_rates,
           W_rel2_rev, b_rel2_rev, W_root2_rev,
           W_post, b_post, rel_emb):
    npe = EPAD - E
    pad_src = jnp.arange(npe, dtype=jnp.int32) % N
    pad_dst = N + (jnp.arange(npe, dtype=jnp.int32) % 8)

    (tab1, init1, src1, src2, dst, ssrc, sdst, relm) = _dense(
        x_user, x_item, W_rel1_rates, W_root1_rates, W_rel1_rev, W_root1_rev,
        W_rel2_rates, W_root2_rates, W_rel2_rev, W_root2_rev, W_post,
        b_rel1_rates.reshape(1, -1), b_rel1_rev.reshape(1, -1),
        b_rel2_rates.reshape(1, -1), b_rel2_rev.reshape(1, -1),
        b_post.reshape(1, -1),
        edge_index_rates, edge_index_rev, pad_src, pad_dst, rel_emb)

    q = _seg_pass(tab1, init1, src1, dst)
    p = _seg_pass(q, q, src2, dst)
    return _score(p, ssrc, sdst, relm)


# speedup vs baseline: 29.3659x; 1.0337x over previous
import functools

import jax
import jax.numpy as jnp
from jax import lax
from jax.experimental import pallas as pl
from jax.experimental.pallas import tpu as pltpu
from jax.experimental.pallas import tpu_sc as plsc

N = 5000
NPAD = 5008
E = 160000
NSUB = 16
WIN = 128
NWIN = 80
ET = WIN * NWIN
NBUF = 8
EPAD = ET * NSUB
W = 8
F32 = jnp.float32


def _dense_body(xu_ref, xi_ref, wr1r_ref, wo1r_ref, wr1v_ref, wo1v_ref,
                wr2r_ref, wo2r_ref, wr2v_ref, wo2v_ref, wpost_ref,
                br1r_ref, br1v_ref, br2r_ref, br2v_ref, bpost_ref,
                eir_ref, eiv_ref, pads_ref, padd_ref, rel_ref,
                tab_ref, init_ref, src1_ref, src2_ref, dst_ref,
                ssrc_ref, sdst_ref, relm_ref):
    er = eir_ref[...]
    ev = eiv_ref[...]
    ps = pads_ref[...]
    pd = padd_ref[...]
    sr = jnp.concatenate([er[0], ps])
    dr = jnp.concatenate([er[1], pd])
    sv = jnp.concatenate([ev[0], ps])
    dv = jnp.concatenate([ev[1], pd])
    src1_ref[...] = jnp.stack([sr, sv + NPAD]).reshape(2, NSUB, NWIN, WIN)
    src2_ref[...] = jnp.stack([sr + NPAD, sv]).reshape(2, NSUB, NWIN, WIN)
    dst_ref[...] = jnp.stack([dr, dv]).reshape(2, NSUB, NWIN, WIN)
    ssrc_ref[...] = jnp.stack(
        [(sr + NPAD) * W, sv * W + 2]).reshape(2, NSUB, ET)
    sdst_ref[...] = jnp.stack(
        [dr * W + 2, (dv + NPAD) * W]).reshape(2, NSUB, ET)
    relm_ref[...] = jnp.broadcast_to(
        rel_ref[...][:, :, None], (2, 2, 16)).reshape(2, 32)

    dot = functools.partial(jnp.dot, precision=lax.Precision.HIGHEST,
                            preferred_element_type=F32)
    wp = wpost_ref[...]
    g = dot(wr2v_ref[...], wp)
    t = dot(wo2r_ref[...], wp)
    gp = dot(wr2r_ref[...], wp)
    tp = dot(wo2v_ref[...], wp)
    zpad = jnp.zeros((128, 4), F32)
    gt = jnp.concatenate([g, t, zpad], axis=1)
    tg = jnp.concatenate([tp, gp, zpad], axis=1)
    c_item = dot(br2r_ref[...], wp) + bpost_ref[...]
    c_user = dot(br2v_ref[...], wp) + bpost_ref[...]
    zero2 = jnp.zeros((1, 2), F32)
    zero4 = jnp.zeros((1, 4), F32)
    bias_i = dot(br1r_ref[...], gt) + jnp.concatenate(
        [zero2, c_item, zero4], axis=1)
    bias_u = dot(br1v_ref[...], tg) + jnp.concatenate(
        [c_user, zero2, zero4], axis=1)
    mu = jnp.concatenate([dot(wr1r_ref[...], gt),
                          dot(wo1v_ref[...], tg)], axis=1)
    mi = jnp.concatenate([dot(wr1v_ref[...], tg),
                          dot(wo1r_ref[...], gt)], axis=1)
    ou = dot(xu_ref[...], mu)
    oi = dot(xi_ref[...], mi)
    zrows = jnp.zeros((NPAD - N, W), F32)
    tab_ref[0:N] = ou[:, 0:W]
    tab_ref[N:NPAD] = zrows
    tab_ref[NPAD:NPAD + N] = oi[:, 0:W]
    tab_ref[NPAD + N:] = zrows
    init_ref[0:N] = oi[:, W:] + bias_i
    init_ref[N:NPAD] = zrows
    init_ref[NPAD:NPAD + N] = ou[:, W:] + bias_u
    init_ref[NPAD + N:] = zrows


def _dense(xu, xi, wr1r, wo1r, wr1v, wo1v, wr2r, wo2r, wr2v, wo2v, wpost,
           br1r, br1v, br2r, br2v, bpost, eir, eiv, pads, padd, rel):
    i32 = jnp.int32
    outs = [jax.ShapeDtypeStruct((2 * NPAD, W), F32),
            jax.ShapeDtypeStruct((2 * NPAD, W), F32),
            jax.ShapeDtypeStruct((2, NSUB, NWIN, WIN), i32),
            jax.ShapeDtypeStruct((2, NSUB, NWIN, WIN), i32),
            jax.ShapeDtypeStruct((2, NSUB, NWIN, WIN), i32),
            jax.ShapeDtypeStruct((2, NSUB, ET), i32),
            jax.ShapeDtypeStruct((2, NSUB, ET), i32),
            jax.ShapeDtypeStruct((2, 32), F32)]
    return pl.pallas_call(_dense_body, out_shape=outs)(
        xu, xi, wr1r, wo1r, wr1v, wo1v, wr2r, wo2r, wr2v, wo2v, wpost,
        br1r, br1v, br2r, br2v, bpost, eir, eiv, pads, padd, rel)


_MESH = plsc.VectorSubcoreMesh(core_axis_name="c", subcore_axis_name="s")
_SC_PARAMS = pltpu.CompilerParams(needs_layout_passes=False,
                                  use_tc_tiling_on_sc=False)


@functools.partial(
    pl.kernel, mesh=_MESH,
    out_type=jax.ShapeDtypeStruct((2 * NPAD, W), F32),
    scratch_types=[
        pltpu.VMEM_SHARED((NPAD, W), F32),
        pltpu.VMEM((NWIN, WIN), jnp.int32),
        pltpu.VMEM((NWIN, WIN), jnp.int32),
        pltpu.VMEM((NBUF, WIN, W), F32),
        pltpu.VMEM((NPAD, W), F32),
        pltpu.SemaphoreType.DMA((NBUF,)),
        pltpu.SemaphoreType.DMA((NBUF,)),
    ],
    compiler_params=_SC_PARAMS,
)
def _seg_pass(tab_hbm, init_hbm, src_hbm, dst_hbm, out_hbm,
              acc_sp, sidx_v, didx_v, rows_v, bounce_v, gsem, ssem):
    c = lax.axis_index("c")
    s = lax.axis_index("s")

    @pl.when(s == 0)
    def _stage_init():
        pltpu.sync_copy(init_hbm.at[pl.ds(c * NPAD, NPAD)], bounce_v)
        pltpu.sync_copy(bounce_v, acc_sp)

    pltpu.sync_copy(src_hbm.at[c, s], sidx_v)
    pltpu.sync_copy(dst_hbm.at[c, s], didx_v)
    plsc.subcore_barrier()

    def block(i, carry):
        base = i * NBUF
        gs = [pltpu.async_copy(tab_hbm.at[sidx_v.at[base + k]],
                               rows_v.at[k], gsem.at[k])
              for k in range(NBUF)]
        scs = []
        for k in range(NBUF):
            gs[k].wait()
            scs.append(pltpu.async_copy(
                rows_v.at[k], acc_sp.at[didx_v.at[base + k]], ssem.at[k],
                add=True))
        for cp in scs:
            cp.wait()
        return carry

    lax.fori_loop(0, NWIN // NBUF, block, 0)
    plsc.subcore_barrier()

    @pl.when(s == 0)
    def _flush():
        pltpu.sync_copy(acc_sp, bounce_v)
        pltpu.sync_copy(bounce_v, out_hbm.at[pl.ds(c * NPAD, NPAD)])


ETT = E - (NSUB - 1) * ET


@functools.partial(
    pl.kernel, mesh=_MESH,
    out_type=jax.ShapeDtypeStruct((2 * E,), F32),
    scratch_types=[
        pltpu.VMEM((2 * NPAD, W), F32),
        pltpu.VMEM((ET,), jnp.int32),
        pltpu.VMEM((ET,), jnp.int32),
        pltpu.VMEM((ET,), F32),
        pltpu.VMEM((32,), F32),
        pltpu.VMEM_SHARED((2 * NPAD, W), F32),
    ],
    compiler_params=_SC_PARAMS,
)
def _score(ptab_hbm, src_hbm, dst_hbm, rel_hbm, out_hbm,
           tab_v, sidx_v, didx_v, sbuf_v, rel_v, shr_sp):
    c = lax.axis_index("c")
    s = lax.axis_index("s")
    chunk = (2 * NPAD) // NSUB
    seg = pl.ds(s * chunk, chunk)
    pltpu.sync_copy(ptab_hbm.at[seg], tab_v.at[seg])
    pltpu.sync_copy(tab_v.at[seg], shr_sp.at[seg])
    pltpu.sync_copy(src_hbm.at[c, s], sidx_v)
    pltpu.sync_copy(dst_hbm.at[c, s], didx_v)
    pltpu.sync_copy(rel_hbm.at[c], rel_v)
    plsc.subcore_barrier()
    pltpu.sync_copy(shr_sp, tab_v)
    r0 = rel_v[pl.ds(0, 16)]
    r1 = rel_v[pl.ds(16, 16)]

    def group(g, carry):
        sl = pl.ds(g * 16, 16)
        si = sidx_v[sl]
        di = didx_v[sl]
        sro = lax.shift_right_logical(si, 3)
        sco = jnp.bitwise_and(si, 7)
        dro = lax.shift_right_logical(di, 3)
        dco = jnp.bitwise_and(di, 7)
        a0 = plsc.load_gather(tab_v, [sro, sco])
        a1 = plsc.load_gather(tab_v, [sro, sco + 1])
        b0 = plsc.load_gather(tab_v, [dro, dco])
        b1 = plsc.load_gather(tab_v, [dro, dco + 1])
        sbuf_v[sl] = a0 * b0 * r0 + a1 * b1 * r1
        return carry

    lax.fori_loop(0, ET // 16, group, 0)
    base = c * E + s * ET

    @pl.when(s < NSUB - 1)
    def _full():
        pltpu.sync_copy(sbuf_v, out_hbm.at[pl.ds(base, ET)])

    @pl.when(s == NSUB - 1)
    def _tail():
        pltpu.sync_copy(sbuf_v.at[pl.ds(0, ETT)], out_hbm.at[pl.ds(base, ETT)])


def kernel(x_user, x_item, edge_index_rates, edge_index_rev, snap,
           W_rel1_rates, b_rel1_rates, W_root1_rates,
           W_rel1_rev, b_rel1_rev, W_root1_rev,
           W_rel2_rates, b_rel2_rates, W_root2_rates,
           W_rel2_rev, b_rel2_rev, W_root2_rev,
           W_post, b_post, rel_emb):
    npe = EPAD - E
    pad_src = jnp.arange(npe, dtype=jnp.int32) % N
    pad_dst = N + (jnp.arange(npe, dtype=jnp.int32) % 8)

    (tab1, init1, src1, src2, dst, ssrc, sdst, relm) = _dense(
        x_user, x_item, W_rel1_rates, W_root1_rates, W_rel1_rev, W_root1_rev,
        W_rel2_rates, W_root2_rates, W_rel2_rev, W_root2_rev, W_post,
        b_rel1_rates.reshape(1, -1), b_rel1_rev.reshape(1, -1),
        b_rel2_rates.reshape(1, -1), b_rel2_rev.reshape(1, -1),
        b_post.reshape(1, -1),
        edge_index_rates, edge_index_rev, pad_src, pad_dst, rel_emb)

    q = _seg_pass(tab1, init1, src1, dst)
    p = _seg_pass(q, q, src2, dst)
    return _score(p, ssrc, sdst, relm)
